# baseline (device time: 61373 ns/iter reference)
import jax
import jax.numpy as jnp
from jax import lax
from jax.experimental import pallas as pl
from jax.experimental.pallas import tpu as pltpu

B, NB, BS, H, D = 8, 64, 16, 8, 64
P_LOCAL = 64
T = P_LOCAL * BS
BH = B * H
HD = H * D
NEG = -1e30


def kernel(Q, K, V, bt, lens):
    qb = Q[:, 0].astype(jnp.bfloat16)
    eye = jnp.eye(H, dtype=jnp.bfloat16)
    q_bh = (qb[:, :, None, :] * eye[None, :, :, None]).reshape(BH, HD)
    k2t = K.reshape(T, HD).astype(jnp.bfloat16).T
    v2 = V.reshape(T, HD).astype(jnp.bfloat16)
    lens2d = lens.reshape(B, 1)

    def body(q_ref, k_ref, v_ref, bt_ref, lens_ref, out_ref,
             send_o, recv_o, send_ml, recv_ml, send_sems, recv_sems):
        my_x = lax.axis_index("x")
        my_y = lax.axis_index("y")
        nbr = (my_x, 1 - my_y)

        barrier = pltpu.get_barrier_semaphore()
        pl.semaphore_signal(barrier, inc=1, device_id=nbr,
                            device_id_type=pl.DeviceIdType.MESH)
        pl.semaphore_wait(barrier, 1)

        bt_v = bt_ref[:, :]
        slot = lax.broadcasted_iota(jnp.int32, (B, NB), 1)
        bt_eff = jnp.where(slot < lens_ref[:, :], bt_v, -1)
        bt_b = lax.broadcast_in_dim(bt_eff, (B, T, NB), (0, 2))
        page_of_t = (lax.broadcasted_iota(jnp.int32, (B, T, NB), 1) // BS
                     + my_y * P_LOCAL)
        counts_t = jnp.sum((bt_b == page_of_t).astype(jnp.float32),
                           axis=2)
        counts_bh = lax.broadcast_in_dim(
            counts_t, (B, H, T), (0, 2)).reshape(BH, T)
        has = counts_bh > 0.0

        s = lax.dot_general(
            q_ref[...], k_ref[...], (((1,), (0,)), ((), ())),
            preferred_element_type=jnp.float32) * (D ** -0.5)
        s = jnp.where(has, s, NEG)
        m_l = jnp.max(s, axis=1, keepdims=True)
        p = counts_bh * jnp.exp(s - m_l)
        l_l = jnp.sum(p, axis=1, keepdims=True)

        o_full = lax.dot_general(
            p.astype(jnp.bfloat16), v_ref[...], (((1,), (0,)), ((), ())),
            preferred_element_type=jnp.float32)
        row_h = lax.broadcasted_iota(jnp.int32, (BH, HD), 0) % H
        col_h = lax.broadcasted_iota(jnp.int32, (BH, HD), 1) // D
        o_masked = jnp.where(row_h == col_h, o_full, 0.0)
        o_l = o_masked[:, 0:D]
        for g in range(1, H):
            o_l = o_l + o_masked[:, g * D:(g + 1) * D]

        send_o[...] = o_l
        send_ml[:, 0:1] = m_l
        send_ml[:, 1:2] = l_l

        copy_o = pltpu.make_async_remote_copy(
            src_ref=send_o, dst_ref=recv_o,
            send_sem=send_sems.at[0], recv_sem=recv_sems.at[0],
            device_id=nbr, device_id_type=pl.DeviceIdType.MESH)
        copy_ml = pltpu.make_async_remote_copy(
            src_ref=send_ml, dst_ref=recv_ml,
            send_sem=send_sems.at[1], recv_sem=recv_sems.at[1],
            device_id=nbr, device_id_type=pl.DeviceIdType.MESH)
        copy_o.start()
        copy_ml.start()
        copy_o.wait()
        copy_ml.wait()

        m_o = recv_ml[:, 0:1]
        l_o = recv_ml[:, 1:2]
        o_o = recv_o[...]
        m_g = jnp.maximum(m_l, m_o)
        a = jnp.exp(m_l - m_g)
        b = jnp.exp(m_o - m_g)
        l_g = l_l * a + l_o * b
        out_ref[...] = (o_l * a + o_o * b) / l_g

    out = pl.pallas_call(
        body,
        out_shape=jax.ShapeDtypeStruct((BH, D), jnp.float32),
        in_specs=[pl.BlockSpec(memory_space=pltpu.VMEM)] * 5,
        out_specs=pl.BlockSpec(memory_space=pltpu.VMEM),
        scratch_shapes=[
            pltpu.VMEM((BH, D), jnp.float32),
            pltpu.VMEM((BH, D), jnp.float32),
            pltpu.VMEM((BH, 2), jnp.float32),
            pltpu.VMEM((BH, 2), jnp.float32),
            pltpu.SemaphoreType.DMA((2,)),
            pltpu.SemaphoreType.DMA((2,)),
        ],
        compiler_params=pltpu.CompilerParams(
            collective_id=0, vmem_limit_bytes=100 * 1024 * 1024),
    )(q_bh, k2t, v2, bt, lens2d)
    return out.reshape(B, 1, H, D)


# device time: 14581 ns/iter; 4.2091x vs baseline; 4.2091x over previous
import jax
import jax.numpy as jnp
from jax import lax
from jax.experimental import pallas as pl
from jax.experimental.pallas import tpu as pltpu

B, NB, BS, H, D = 8, 64, 16, 8, 64
HH = H // 2
P_LOCAL = 64
T = P_LOCAL * BS
NEG = -1e30


def kernel(Q, K, V, bt, lens):
    PH = P_LOCAL // 2
    TH = PH * BS

    def body(q_hbm, k_hbm, v_hbm, bt_hbm, lens_ref, out_hbm,
             q_vmem, bt_vmem, out_vmem, k_vmem, v_vmem,
             send_pk, recv_y, recv_x, recv_d,
             load_sems, send_sems, recv_sems):
        my_x = lax.axis_index("x")
        my_y = lax.axis_index("y")
        yn = (my_x, 1 - my_y)
        xn = (1 - my_x, my_y)
        dg = (1 - my_x, 1 - my_y)
        h0 = my_x * HH

        copy_q = pltpu.make_async_copy(q_hbm, q_vmem, load_sems.at[4])
        copy_bt = pltpu.make_async_copy(bt_hbm, bt_vmem, load_sems.at[5])
        copy_q.start()
        copy_bt.start()

        hs = pl.ds(h0, HH)
        copy_k0 = pltpu.make_async_copy(
            k_hbm.at[0:PH, :, hs, :], k_vmem.at[0:PH], load_sems.at[0])
        copy_k1 = pltpu.make_async_copy(
            k_hbm.at[PH:P_LOCAL, :, hs, :], k_vmem.at[PH:P_LOCAL],
            load_sems.at[1])
        copy_v0 = pltpu.make_async_copy(
            v_hbm.at[0:PH, :, hs, :], v_vmem.at[0:PH], load_sems.at[2])
        copy_v1 = pltpu.make_async_copy(
            v_hbm.at[PH:P_LOCAL, :, hs, :], v_vmem.at[PH:P_LOCAL],
            load_sems.at[3])
        copy_k0.start()
        copy_k1.start()
        copy_v0.start()
        copy_v1.start()

        barrier = pltpu.get_barrier_semaphore()
        for peer in (yn, xn, dg):
            pl.semaphore_signal(barrier, inc=1, device_id=peer,
                                device_id_type=pl.DeviceIdType.MESH)
        pl.semaphore_wait(barrier, 3)

        copy_bt.wait()
        lens_mat = jnp.concatenate(
            [jnp.full((1, NB), lens_ref[b], jnp.int32) for b in range(B)],
            axis=0)
        slot = lax.broadcasted_iota(jnp.int32, (B, NB), 1)
        bt_eff = jnp.where(slot < lens_mat, bt_vmem[:, :], -1)
        bt_3 = lax.broadcast_in_dim(bt_eff, (B, P_LOCAL, NB), (0, 2))
        page_3 = (lax.broadcasted_iota(jnp.int32, (B, P_LOCAL, NB), 1)
                  + my_y * P_LOCAL)
        counts_p = jnp.sum((bt_3 == page_3).astype(jnp.float32),
                           axis=2)
        pg = lax.broadcasted_iota(jnp.int32, (P_LOCAL, T), 0)
        tk = lax.broadcasted_iota(jnp.int32, (P_LOCAL, T), 1) // BS
        expand = (pg == tk).astype(jnp.bfloat16)
        counts_t = lax.dot_general(
            counts_p.astype(jnp.bfloat16), expand,
            (((1,), (0,)), ((), ())),
            preferred_element_type=jnp.float32)
        has = counts_t > 0.0

        scale = D ** -0.5

        copy_q.wait()
        is_x0 = my_x == 0
        q_hs = []
        for i in range(HH):
            q_i = jnp.where(is_x0, q_vmem[:, 0, i, :],
                            q_vmem[:, 0, HH + i, :])
            q_hs.append((q_i * scale).astype(jnp.bfloat16))

        copy_k0.wait()
        s0_hs = []
        for i in range(HH):
            k_i = k_vmem[0:PH, :, i, :].reshape(TH, D).astype(jnp.bfloat16)
            s0_hs.append(lax.dot_general(
                q_hs[i], k_i, (((1,), (1,)), ((), ())),
                preferred_element_type=jnp.float32))

        copy_k1.wait()
        p_hs, m_hs, l_hs = [], [], []
        for i in range(HH):
            k_i = (k_vmem[PH:P_LOCAL, :, i, :]
                   .reshape(TH, D).astype(jnp.bfloat16))
            s1_i = lax.dot_general(
                q_hs[i], k_i, (((1,), (1,)), ((), ())),
                preferred_element_type=jnp.float32)
            s_i = jnp.concatenate([s0_hs[i], s1_i], axis=1)
            s_i = jnp.where(has, s_i, NEG)
            m_i = jnp.max(s_i, axis=1, keepdims=True)
            p_i = counts_t * jnp.exp(s_i - m_i)
            l_i = jnp.sum(p_i, axis=1, keepdims=True)
            p_hs.append(p_i.astype(jnp.bfloat16))
            m_hs.append(m_i)
            l_hs.append(l_i)

        copy_v0.wait()
        copy_v1.wait()
        o_hs = []
        for i in range(HH):
            v_i = v_vmem[:, :, i, :].reshape(T, D).astype(jnp.bfloat16)
            o_i = lax.dot_general(
                p_hs[i], v_i, (((1,), (0,)), ((), ())),
                preferred_element_type=jnp.float32)
            send_pk[:, i, 0:D] = o_i
            send_pk[:, i, D:D + 1] = m_hs[i]
            send_pk[:, i, D + 1:D + 2] = l_hs[i]
            o_hs.append(o_i)

        rdmas = []
        for j, (peer, dst) in enumerate(
                [(yn, recv_y), (xn, recv_x), (dg, recv_d)]):
            r = pltpu.make_async_remote_copy(
                src_ref=send_pk, dst_ref=dst,
                send_sem=send_sems.at[j], recv_sem=recv_sems.at[j],
                device_id=peer, device_id_type=pl.DeviceIdType.MESH)
            r.start()
            rdmas.append(r)
        for r in rdmas:
            r.wait()

        mine = []
        for i in range(HH):
            m_o = recv_y[:, i, D:D + 1]
            l_o = recv_y[:, i, D + 1:D + 2]
            o_o = recv_y[:, i, 0:D]
            m_g = jnp.maximum(m_hs[i], m_o)
            a = jnp.exp(m_hs[i] - m_g)
            b = jnp.exp(m_o - m_g)
            mine.append((o_hs[i] * a + o_o * b)
                        / (l_hs[i] * a + l_o * b))

        other = []
        for i in range(HH):
            m_1, l_1 = recv_x[:, i, D:D + 1], recv_x[:, i, D + 1:D + 2]
            m_2, l_2 = recv_d[:, i, D:D + 1], recv_d[:, i, D + 1:D + 2]
            m_g = jnp.maximum(m_1, m_2)
            a = jnp.exp(m_1 - m_g)
            b = jnp.exp(m_2 - m_g)
            other.append((recv_x[:, i, 0:D] * a + recv_d[:, i, 0:D] * b)
                         / (l_1 * a + l_2 * b))

        for i in range(HH):
            out_vmem[:, 0, i, :] = jnp.where(is_x0, mine[i], other[i])
            out_vmem[:, 0, HH + i, :] = jnp.where(is_x0, other[i], mine[i])

        copy_out = pltpu.make_async_copy(out_vmem, out_hbm, load_sems.at[6])
        copy_out.start()
        copy_out.wait()

    return pl.pallas_call(
        body,
        out_shape=jax.ShapeDtypeStruct((B, 1, H, D), jnp.float32),
        in_specs=[
            pl.BlockSpec(memory_space=pl.ANY),
            pl.BlockSpec(memory_space=pl.ANY),
            pl.BlockSpec(memory_space=pl.ANY),
            pl.BlockSpec(memory_space=pl.ANY),
            pl.BlockSpec(memory_space=pltpu.SMEM),
        ],
        out_specs=pl.BlockSpec(memory_space=pl.ANY),
        scratch_shapes=[
            pltpu.VMEM((B, 1, H, D), jnp.float32),
            pltpu.VMEM((B, NB), jnp.int32),
            pltpu.VMEM((B, 1, H, D), jnp.float32),
            pltpu.VMEM((P_LOCAL, BS, HH, D), jnp.float32),
            pltpu.VMEM((P_LOCAL, BS, HH, D), jnp.float32),
            pltpu.VMEM((B, HH, D + 2), jnp.float32),
            pltpu.VMEM((B, HH, D + 2), jnp.float32),
            pltpu.VMEM((B, HH, D + 2), jnp.float32),
            pltpu.VMEM((B, HH, D + 2), jnp.float32),
            pltpu.SemaphoreType.DMA((7,)),
            pltpu.SemaphoreType.DMA((3,)),
            pltpu.SemaphoreType.DMA((3,)),
        ],
        compiler_params=pltpu.CompilerParams(
            collective_id=0, vmem_limit_bytes=100 * 1024 * 1024),
    )(Q, K, V, bt, lens)


# device time: 14243 ns/iter; 4.3090x vs baseline; 1.0237x over previous
import jax
import jax.numpy as jnp
from jax import lax
from jax.experimental import pallas as pl
from jax.experimental.pallas import tpu as pltpu

B, NB, BS, H, D = 8, 64, 16, 8, 64
HH = H // 2
P_LOCAL = 64
T = P_LOCAL * BS
NEG = -1e30


def kernel(Q, K, V, bt, lens):
    PH = P_LOCAL // 2
    TH = PH * BS

    def body(q_hbm, k_hbm, v_hbm, bt_hbm, lens_ref, out_hbm,
             q_vmem, bt_vmem, out_vmem, k_vmem, v_vmem,
             send_pk, recv_y, recv_x, recv_d,
             load_sems, send_sems, recv_sems):
        my_x = lax.axis_index("x")
        my_y = lax.axis_index("y")
        yn = (my_x, 1 - my_y)
        xn = (1 - my_x, my_y)
        dg = (1 - my_x, 1 - my_y)
        h0 = my_x * HH

        copy_q = pltpu.make_async_copy(q_hbm, q_vmem, load_sems.at[4])
        copy_bt = pltpu.make_async_copy(bt_hbm, bt_vmem, load_sems.at[5])
        copy_q.start()
        copy_bt.start()

        hs = pl.ds(h0, HH)
        copy_k0 = pltpu.make_async_copy(
            k_hbm.at[0:PH, :, hs, :], k_vmem.at[0:PH], load_sems.at[0])
        copy_k1 = pltpu.make_async_copy(
            k_hbm.at[PH:P_LOCAL, :, hs, :], k_vmem.at[PH:P_LOCAL],
            load_sems.at[1])
        copy_v0 = pltpu.make_async_copy(
            v_hbm.at[0:PH, :, hs, :], v_vmem.at[0:PH], load_sems.at[2])
        copy_v1 = pltpu.make_async_copy(
            v_hbm.at[PH:P_LOCAL, :, hs, :], v_vmem.at[PH:P_LOCAL],
            load_sems.at[3])
        copy_k0.start()
        copy_k1.start()
        copy_v0.start()
        copy_v1.start()

        barrier = pltpu.get_barrier_semaphore()
        for peer in (yn, xn, dg):
            pl.semaphore_signal(barrier, inc=1, device_id=peer,
                                device_id_type=pl.DeviceIdType.MESH)
        pl.semaphore_wait(barrier, 3)

        copy_bt.wait()
        lens_mat = jnp.concatenate(
            [jnp.full((1, NB), lens_ref[b], jnp.int32) for b in range(B)],
            axis=0)
        slot = lax.broadcasted_iota(jnp.int32, (B, NB), 1)
        bt_eff = jnp.where(slot < lens_mat, bt_vmem[:, :], -1)
        bt_3 = lax.broadcast_in_dim(bt_eff, (B, P_LOCAL, NB), (0, 2))
        page_3 = (lax.broadcasted_iota(jnp.int32, (B, P_LOCAL, NB), 1)
                  + my_y * P_LOCAL)
        counts_p = jnp.sum((bt_3 == page_3).astype(jnp.float32),
                           axis=2)
        pg = lax.broadcasted_iota(jnp.int32, (P_LOCAL, T), 0)
        tk = lax.broadcasted_iota(jnp.int32, (P_LOCAL, T), 1) // BS
        expand = (pg == tk).astype(jnp.bfloat16)
        counts_t = lax.dot_general(
            counts_p.astype(jnp.bfloat16), expand,
            (((1,), (0,)), ((), ())),
            preferred_element_type=jnp.float32)
        has = counts_t > 0.0

        scale = D ** -0.5

        copy_q.wait()
        is_x0 = my_x == 0
        q_hs = []
        for i in range(HH):
            q_i = jnp.where(is_x0, q_vmem[:, 0, i, :],
                            q_vmem[:, 0, HH + i, :])
            q_hs.append((q_i * scale).astype(jnp.bfloat16))

        copy_k0.wait()
        s0_hs = []
        for i in range(HH):
            k_i = k_vmem[0:PH, :, i, :].reshape(TH, D).astype(jnp.bfloat16)
            s0_hs.append(lax.dot_general(
                q_hs[i], k_i, (((1,), (1,)), ((), ())),
                preferred_element_type=jnp.float32))

        copy_k1.wait()
        p_hs, m_hs, l_hs = [], [], []
        for i in range(HH):
            k_i = (k_vmem[PH:P_LOCAL, :, i, :]
                   .reshape(TH, D).astype(jnp.bfloat16))
            s1_i = lax.dot_general(
                q_hs[i], k_i, (((1,), (1,)), ((), ())),
                preferred_element_type=jnp.float32)
            s_i = jnp.concatenate([s0_hs[i], s1_i], axis=1)
            s_i = jnp.where(has, s_i, NEG)
            m_i = jnp.max(s_i, axis=1, keepdims=True)
            p_i = counts_t * jnp.exp(s_i - m_i)
            l_i = jnp.sum(p_i, axis=1, keepdims=True)
            p_hs.append(p_i.astype(jnp.bfloat16))
            m_hs.append(m_i)
            l_hs.append(l_i)

        copy_v0.wait()
        copy_v1.wait()
        o_hs = []
        for i in range(HH):
            v_i = v_vmem[:, :, i, :].reshape(T, D).astype(jnp.bfloat16)
            o_i = lax.dot_general(
                p_hs[i], v_i, (((1,), (0,)), ((), ())),
                preferred_element_type=jnp.float32)
            send_pk[:, i, 0:D] = o_i
            send_pk[:, i, D:D + 1] = m_hs[i]
            send_pk[:, i, D + 1:D + 2] = l_hs[i]
            o_hs.append(o_i)

        rdmas = []
        for j, (peer, dst) in enumerate(
                [(yn, recv_y), (xn, recv_x), (dg, recv_d)]):
            r = pltpu.make_async_remote_copy(
                src_ref=send_pk, dst_ref=dst,
                send_sem=send_sems.at[j], recv_sem=recv_sems.at[j],
                device_id=peer, device_id_type=pl.DeviceIdType.MESH)
            r.start()
            rdmas.append(r)

        rdmas[0].wait()
        mine = []
        for i in range(HH):
            m_o = recv_y[:, i, D:D + 1]
            l_o = recv_y[:, i, D + 1:D + 2]
            o_o = recv_y[:, i, 0:D]
            m_g = jnp.maximum(m_hs[i], m_o)
            a = jnp.exp(m_hs[i] - m_g)
            b = jnp.exp(m_o - m_g)
            mine.append((o_hs[i] * a + o_o * b)
                        / (l_hs[i] * a + l_o * b))

        rdmas[1].wait()
        rdmas[2].wait()
        other = []
        for i in range(HH):
            m_1, l_1 = recv_x[:, i, D:D + 1], recv_x[:, i, D + 1:D + 2]
            m_2, l_2 = recv_d[:, i, D:D + 1], recv_d[:, i, D + 1:D + 2]
            m_g = jnp.maximum(m_1, m_2)
            a = jnp.exp(m_1 - m_g)
            b = jnp.exp(m_2 - m_g)
            other.append((recv_x[:, i, 0:D] * a + recv_d[:, i, 0:D] * b)
                         / (l_1 * a + l_2 * b))

        for i in range(HH):
            out_vmem[:, 0, i, :] = jnp.where(is_x0, mine[i], other[i])
            out_vmem[:, 0, HH + i, :] = jnp.where(is_x0, other[i], mine[i])

        copy_out = pltpu.make_async_copy(out_vmem, out_hbm, load_sems.at[6])
        copy_out.start()
        copy_out.wait()

    return pl.pallas_call(
        body,
        out_shape=jax.ShapeDtypeStruct((B, 1, H, D), jnp.float32),
        in_specs=[
            pl.BlockSpec(memory_space=pl.ANY),
            pl.BlockSpec(memory_space=pl.ANY),
            pl.BlockSpec(memory_space=pl.ANY),
            pl.BlockSpec(memory_space=pl.ANY),
            pl.BlockSpec(memory_space=pltpu.SMEM),
        ],
        out_specs=pl.BlockSpec(memory_space=pl.ANY),
        scratch_shapes=[
            pltpu.VMEM((B, 1, H, D), jnp.float32),
            pltpu.VMEM((B, NB), jnp.int32),
            pltpu.VMEM((B, 1, H, D), jnp.float32),
            pltpu.VMEM((P_LOCAL, BS, HH, D), jnp.float32),
            pltpu.VMEM((P_LOCAL, BS, HH, D), jnp.float32),
            pltpu.VMEM((B, HH, D + 2), jnp.float32),
            pltpu.VMEM((B, HH, D + 2), jnp.float32),
            pltpu.VMEM((B, HH, D + 2), jnp.float32),
            pltpu.VMEM((B, HH, D + 2), jnp.float32),
            pltpu.SemaphoreType.DMA((7,)),
            pltpu.SemaphoreType.DMA((3,)),
            pltpu.SemaphoreType.DMA((3,)),
        ],
        compiler_params=pltpu.CompilerParams(
            collective_id=0, vmem_limit_bytes=100 * 1024 * 1024),
    )(Q, K, V, bt, lens)
